# baseline (speedup 1.0000x reference)
import functools

import jax
import jax.numpy as jnp
from jax import lax
from jax.experimental import pallas as pl
from jax.experimental.pallas import tpu as pltpu
from jax.experimental.pallas import tpu_sc as plsc

EMB = 64
SCALE = 8.0
NC = 2
NS = 16
L = 16
NW = NC * NS
CHUNK = 128


@functools.lru_cache(maxsize=None)
def _make(n_chunks):
    mesh = plsc.VectorSubcoreMesh(
        core_axis_name="c", subcore_axis_name="s",
        num_cores=NC, num_subcores=NS)

    def body(tok_hbm, table_hbm, out_hbm, idx_v, rows_v, sem):
        wid = lax.axis_index("s") * NC + lax.axis_index("c")
        pltpu.sync_copy(tok_hbm.at[wid], idx_v)

        def step(j, carry):
            pltpu.async_copy(table_hbm.at[idx_v.at[j]], rows_v, sem).wait()

            def scale_row(i, c):
                for k in range(EMB // L):
                    s = pl.ds(k * L, L)
                    rows_v[i, s] = rows_v[i, s] * SCALE
                return c

            lax.fori_loop(0, CHUNK, scale_row, 0)
            pltpu.sync_copy(rows_v, out_hbm.at[wid, j])
            return carry

        lax.fori_loop(0, n_chunks, step, 0)

    return pl.kernel(
        body,
        out_type=jax.ShapeDtypeStruct((NW, n_chunks, CHUNK, EMB), jnp.float32),
        mesh=mesh,
        compiler_params=pltpu.CompilerParams(use_tc_tiling_on_sc=False),
        scratch_types=[
            pltpu.VMEM((n_chunks, CHUNK), jnp.int32),
            pltpu.VMEM((CHUNK, EMB), jnp.float32),
            pltpu.SemaphoreType.DMA,
        ],
    )


def kernel(tokens, table):
    bt, sl = tokens.shape
    b = bt * sl
    n_chunks = b // (NW * CHUNK)
    tok = tokens.reshape(NW, n_chunks, CHUNK).astype(jnp.int32)
    out = _make(n_chunks)(tok, table)
    return out.reshape(bt, sl, EMB)


# speedup vs baseline: 1.2072x; 1.2072x over previous
import functools

import jax
import jax.numpy as jnp
from jax import lax
from jax.experimental import pallas as pl
from jax.experimental.pallas import tpu as pltpu
from jax.experimental.pallas import tpu_sc as plsc

EMB = 64
SCALE = 8.0
NC = 2
NS = 16
L = 16
NW = NC * NS
CHUNK = 128
K = 4
GROUP = K * CHUNK


@functools.lru_cache(maxsize=None)
def _make(n_chunks):
    n_groups = n_chunks // K
    assert n_chunks % K == 0 and n_groups % 2 == 0 and n_groups >= 4
    mesh = plsc.VectorSubcoreMesh(
        core_axis_name="c", subcore_axis_name="s",
        num_cores=NC, num_subcores=NS)

    def body(tok_hbm, table_hbm, out_hbm, idx_v, rows0, rows1, sem0, sem1):
        wid = lax.axis_index("s") * NC + lax.axis_index("c")
        pltpu.sync_copy(tok_hbm.at[wid], idx_v)

        def fire(g, buf, sem):
            for b in range(K):
                pltpu.async_copy(
                    table_hbm.at[idx_v.at[g * K + b]],
                    buf.at[pl.ds(b * CHUNK, CHUNK)], sem)

        def drain(g, buf, sem):
            for b in range(K):
                pltpu.make_async_copy(
                    table_hbm.at[idx_v.at[g * K + b]],
                    buf.at[pl.ds(b * CHUNK, CHUNK)], sem).wait()

        def scale(buf):
            @pl.loop(0, GROUP, unroll=4)
            def _(r):
                for k in range(EMB // L):
                    s = pl.ds(k * L, L)
                    buf[r, s] = buf[r, s] * SCALE

        def process(g, buf, sem, nxt_g, nxt_buf, nxt_sem):
            if nxt_g is not None:
                fire(nxt_g, nxt_buf, nxt_sem)
            drain(g, buf, sem)
            scale(buf)
            pltpu.sync_copy(buf, out_hbm.at[wid, pl.ds(g * GROUP, GROUP)])

        fire(0, rows0, sem0)

        @pl.loop(0, n_groups - 2, step=2)
        def _(go):
            process(go, rows0, sem0, go + 1, rows1, sem1)
            process(go + 1, rows1, sem1, go + 2, rows0, sem0)

        process(n_groups - 2, rows0, sem0, n_groups - 1, rows1, sem1)
        process(n_groups - 1, rows1, sem1, None, None, None)

    return pl.kernel(
        body,
        out_type=jax.ShapeDtypeStruct((NW, n_chunks * CHUNK, EMB), jnp.float32),
        mesh=mesh,
        compiler_params=pltpu.CompilerParams(use_tc_tiling_on_sc=False),
        scratch_types=[
            pltpu.VMEM((n_chunks, CHUNK), jnp.int32),
            pltpu.VMEM((GROUP, EMB), jnp.float32),
            pltpu.VMEM((GROUP, EMB), jnp.float32),
            pltpu.SemaphoreType.DMA,
            pltpu.SemaphoreType.DMA,
        ],
    )


def kernel(tokens, table):
    bt, sl = tokens.shape
    b = bt * sl
    n_chunks = b // (NW * CHUNK)
    tok = tokens.reshape(NW, n_chunks, CHUNK).astype(jnp.int32)
    out = _make(n_chunks)(tok, table)
    return out.reshape(bt, sl, EMB)
